# Initial kernel scaffold; baseline (speedup 1.0000x reference)
import jax
import jax.numpy as jnp
from jax.experimental import pallas as pl


def kernel(x, edge_index, W1a, b1a, W1b, b1b, W2a, b2a, W2b, b2b):
    raise NotImplementedError("write your pallas kernel here")


# speedup vs baseline: 2.3160x; 2.3160x over previous
import functools

import jax
import jax.numpy as jnp
from jax import lax
from jax.experimental import pallas as pl
from jax.experimental.pallas import tpu as pltpu
from jax.experimental.pallas import tpu_sc as plsc

N = 10000
D = 128
E = 320000
NC = 2
NS = 16
NW = NC * NS
CH = 128
EPW = 10240
E_PAD = NW * EPW
STEPS = EPW // CH
AGG_ROWS = 10240
ZROWS = AGG_ROWS // NS


def _sc_agg_body(x_hbm, src_hbm, dst_hbm, zeros_hbm, out_hbm,
                 src_v, dst_v, rows_v, agg_sh, sem):
    cid = lax.axis_index("c")
    sid = lax.axis_index("s")

    pltpu.sync_copy(zeros_hbm, agg_sh.at[pl.ds(sid * ZROWS, ZROWS)])
    plsc.subcore_barrier()

    base = (cid * NS + sid) * EPW

    def step(i, carry):
        off = base + i * CH
        pltpu.sync_copy(src_hbm.at[pl.ds(off, CH)], src_v)
        pltpu.sync_copy(dst_hbm.at[pl.ds(off, CH)], dst_v)
        pltpu.async_copy(x_hbm.at[src_v], rows_v, sem).wait()
        pltpu.sync_copy(rows_v, agg_sh.at[dst_v], add=True)
        return carry

    lax.fori_loop(0, STEPS, step, 0)
    plsc.subcore_barrier()

    pltpu.sync_copy(agg_sh.at[pl.ds(sid * ZROWS, ZROWS)],
                    out_hbm.at[cid, pl.ds(sid * ZROWS, ZROWS)])


_sc_agg = pl.kernel(
    _sc_agg_body,
    out_type=jax.ShapeDtypeStruct((NC, AGG_ROWS, D), jnp.float32),
    mesh=plsc.VectorSubcoreMesh(core_axis_name="c", subcore_axis_name="s"),
    scratch_types=[
        pltpu.VMEM((CH,), jnp.int32),
        pltpu.VMEM((CH,), jnp.int32),
        pltpu.VMEM((CH, D), jnp.float32),
        pltpu.VMEM_SHARED((AGG_ROWS, D), jnp.float32),
        pltpu.SemaphoreType.DMA,
    ],
)


def _mlp_body(x_ref, agg_ref, wa_ref, ba_ref, wb_ref, bb_ref, o_ref,
              *, final_relu):
    z = x_ref[...] + agg_ref[0] + agg_ref[1]
    h = jnp.dot(z, wa_ref[...], preferred_element_type=jnp.float32)
    h = jnp.maximum(h + ba_ref[...], 0.0)
    h = jnp.dot(h, wb_ref[...], preferred_element_type=jnp.float32)
    h = h + bb_ref[...]
    if final_relu:
        h = jnp.maximum(h, 0.0)
    o_ref[...] = h


def _mlp(x, agg, wa, ba, wb, bb, final_relu):
    rows = 1000
    grid = (N // rows,)
    k = wb.shape[1]
    return pl.pallas_call(
        functools.partial(_mlp_body, final_relu=final_relu),
        grid=grid,
        in_specs=[
            pl.BlockSpec((rows, D), lambda i: (i, 0)),
            pl.BlockSpec((NC, rows, D), lambda i: (0, i, 0)),
            pl.BlockSpec((D, D), lambda i: (0, 0)),
            pl.BlockSpec((1, D), lambda i: (0, 0)),
            pl.BlockSpec((D, k), lambda i: (0, 0)),
            pl.BlockSpec((1, k), lambda i: (0, 0)),
        ],
        out_specs=pl.BlockSpec((rows, k), lambda i: (i, 0)),
        out_shape=jax.ShapeDtypeStruct((N, k), jnp.float32),
    )(x, agg, wa, ba.reshape(1, -1), wb, bb.reshape(1, -1))


def kernel(x, edge_index, W1a, b1a, W1b, b1b, W2a, b2a, W2b, b2b):
    ei = edge_index.astype(jnp.int32)
    npad = E_PAD - E
    src = jnp.concatenate([ei[0], jnp.zeros((npad,), jnp.int32)])
    pad_dst = N + (jnp.arange(npad, dtype=jnp.int32) % (AGG_ROWS - N))
    dst = jnp.concatenate([ei[1], pad_dst])
    zeros_blk = jnp.zeros((ZROWS, D), jnp.float32)

    agg1 = _sc_agg(x, src, dst, zeros_blk)
    h = _mlp(x, agg1, W1a, b1a, W1b, b1b, final_relu=True)
    agg2 = _sc_agg(h, src, dst, zeros_blk)
    return _mlp(h, agg2, W2a, b2a, W2b, b2b, final_relu=False)


# speedup vs baseline: 3.0070x; 1.2984x over previous
import functools

import jax
import jax.numpy as jnp
from jax import lax
from jax.experimental import pallas as pl
from jax.experimental.pallas import tpu as pltpu
from jax.experimental.pallas import tpu_sc as plsc

N = 10000
D = 128
E = 320000
NC = 2
NS = 16
NW = NC * NS
CH = 64
EPW = 10240
E_PAD = NW * EPW
STEPS = EPW // CH
K = 5
G = STEPS // K
AGG_ROWS = 10240
ZROWS = AGG_ROWS // NS


def _sc_agg_body(x_hbm, eidx_hbm, zeros_hbm, out_hbm,
                 idx_v, rows_v, agg_sh, sem_i, sem_g, sem_s):
    cid = lax.axis_index("c")
    sid = lax.axis_index("s")
    wid = cid * NS + sid

    pltpu.sync_copy(zeros_hbm, agg_sh.at[pl.ds(sid * ZROWS, ZROWS)])
    plsc.subcore_barrier()

    def group(g, carry):
        pltpu.sync_copy(eidx_hbm.at[wid, g], idx_v)
        gcs = [pltpu.async_copy(x_hbm.at[idx_v.at[0, b]],
                                rows_v.at[b], sem_g)
               for b in range(K)]
        for c in gcs:
            c.wait()
        scs = [pltpu.async_copy(rows_v.at[b],
                                agg_sh.at[idx_v.at[1, b]],
                                sem_s, add=True)
               for b in range(K)]
        for c in scs:
            c.wait()
        return carry

    lax.fori_loop(0, G, group, 0)
    plsc.subcore_barrier()

    pltpu.sync_copy(agg_sh.at[pl.ds(sid * ZROWS, ZROWS)],
                    out_hbm.at[cid, pl.ds(sid * ZROWS, ZROWS)])


_sc_agg = pl.kernel(
    _sc_agg_body,
    out_type=jax.ShapeDtypeStruct((NC, AGG_ROWS, D), jnp.float32),
    mesh=plsc.VectorSubcoreMesh(core_axis_name="c", subcore_axis_name="s"),
    scratch_types=[
        pltpu.VMEM((2, K, CH), jnp.int32),
        pltpu.VMEM((K, CH, D), jnp.float32),
        pltpu.VMEM_SHARED((AGG_ROWS, D), jnp.float32),
        pltpu.SemaphoreType.DMA,
        pltpu.SemaphoreType.DMA,
        pltpu.SemaphoreType.DMA,
    ],
)


def _mlp_body(x_ref, agg_ref, wa_ref, ba_ref, wb_ref, bb_ref, o_ref,
              *, final_relu):
    z = x_ref[...] + agg_ref[0] + agg_ref[1]
    h = jnp.dot(z, wa_ref[...], preferred_element_type=jnp.float32)
    h = jnp.maximum(h + ba_ref[...], 0.0)
    h = jnp.dot(h, wb_ref[...], preferred_element_type=jnp.float32)
    h = h + bb_ref[...]
    if final_relu:
        h = jnp.maximum(h, 0.0)
    o_ref[...] = h


def _mlp(x, agg, wa, ba, wb, bb, final_relu):
    rows = 1000
    grid = (N // rows,)
    k = wb.shape[1]
    return pl.pallas_call(
        functools.partial(_mlp_body, final_relu=final_relu),
        grid=grid,
        in_specs=[
            pl.BlockSpec((rows, D), lambda i: (i, 0)),
            pl.BlockSpec((NC, rows, D), lambda i: (0, i, 0)),
            pl.BlockSpec((D, D), lambda i: (0, 0)),
            pl.BlockSpec((1, D), lambda i: (0, 0)),
            pl.BlockSpec((D, k), lambda i: (0, 0)),
            pl.BlockSpec((1, k), lambda i: (0, 0)),
        ],
        out_specs=pl.BlockSpec((rows, k), lambda i: (i, 0)),
        out_shape=jax.ShapeDtypeStruct((N, k), jnp.float32),
    )(x, agg, wa, ba.reshape(1, -1), wb, bb.reshape(1, -1))


def kernel(x, edge_index, W1a, b1a, W1b, b1b, W2a, b2a, W2b, b2b):
    ei = edge_index.astype(jnp.int32)
    npad = E_PAD - E
    src = jnp.concatenate([ei[0], jnp.zeros((npad,), jnp.int32)])
    pad_dst = N + (jnp.arange(npad, dtype=jnp.int32) % (AGG_ROWS - N))
    dst = jnp.concatenate([ei[1], pad_dst])
    eidx = jnp.stack([src.reshape(NW, G, K, CH),
                      dst.reshape(NW, G, K, CH)], axis=2)
    zeros_blk = jnp.zeros((ZROWS, D), jnp.float32)

    agg1 = _sc_agg(x, eidx, zeros_blk)
    h = _mlp(x, agg1, W1a, b1a, W1b, b1b, final_relu=True)
    agg2 = _sc_agg(h, eidx, zeros_blk)
    return _mlp(h, agg2, W2a, b2a, W2b, b2b, final_relu=False)


# speedup vs baseline: 3.2748x; 1.0891x over previous
import functools

import jax
import jax.numpy as jnp
from jax import lax
from jax.experimental import pallas as pl
from jax.experimental.pallas import tpu as pltpu
from jax.experimental.pallas import tpu_sc as plsc

N = 10000
D = 128
E = 320000
NC = 2
NS = 16
NW = NC * NS
CH = 128
EPW = 10240
E_PAD = NW * EPW
STEPS = EPW // CH
HSTEPS = STEPS // 2
NBUF = 2
AGG_ROWS = 10240
ZROWS = AGG_ROWS // NS


def _sc_agg_body(x_hbm, eidx_hbm, zeros_hbm, out_hbm,
                 idx_v, rows_v, agg_sh, sem_i, sem_g, sem_s):
    cid = lax.axis_index("c")
    sid = lax.axis_index("s")
    wid = cid * NS + sid

    pltpu.sync_copy(zeros_hbm, agg_sh.at[pl.ds(sid * ZROWS, ZROWS)])
    plsc.subcore_barrier()

    def gather(i, b):
        return pltpu.async_copy(x_hbm.at[idx_v.at[0, i]], rows_v.at[b],
                                sem_g)

    def gather_wait(i, b):
        pltpu.make_async_copy(x_hbm.at[idx_v.at[0, i]], rows_v.at[b],
                              sem_g).wait()

    def scatter(i, b):
        return pltpu.async_copy(rows_v.at[b], agg_sh.at[idx_v.at[1, i]],
                                sem_s, add=True)

    def scatter_wait(i, b):
        pltpu.make_async_copy(rows_v.at[b], agg_sh.at[idx_v.at[1, i]],
                              sem_s).wait()

    for h in range(2):
        pltpu.sync_copy(eidx_hbm.at[wid, h], idx_v)
        for j in range(NBUF - 1):
            gather(j, j)
        gather_wait(0, 0)
        scatter(0, 0)
        gather(NBUF - 1, NBUF - 1)

        def step(i, carry):
            b = lax.rem(i, NBUF)
            nb = lax.rem(i + NBUF - 1, NBUF)
            gather_wait(i, b)
            scatter(i, b)
            scatter_wait(i - 1, nb)
            gather(i + NBUF - 1, nb)
            return carry

        lax.fori_loop(1, HSTEPS - NBUF + 1, step, 0)
        for e in range(HSTEPS - NBUF + 1, HSTEPS):
            gather_wait(e, e % NBUF)
            scatter(e, e % NBUF)
            scatter_wait(e - 1, (e - 1) % NBUF)
        scatter_wait(HSTEPS - 1, (HSTEPS - 1) % NBUF)

    plsc.subcore_barrier()

    pltpu.sync_copy(agg_sh.at[pl.ds(sid * ZROWS, ZROWS)],
                    out_hbm.at[cid, pl.ds(sid * ZROWS, ZROWS)])


_sc_agg = pl.kernel(
    _sc_agg_body,
    out_type=jax.ShapeDtypeStruct((NC, AGG_ROWS, D), jnp.float32),
    mesh=plsc.VectorSubcoreMesh(core_axis_name="c", subcore_axis_name="s"),
    scratch_types=[
        pltpu.VMEM((2, HSTEPS, CH), jnp.int32),
        pltpu.VMEM((NBUF, CH, D), jnp.float32),
        pltpu.VMEM_SHARED((AGG_ROWS, D), jnp.float32),
        pltpu.SemaphoreType.DMA,
        pltpu.SemaphoreType.DMA,
        pltpu.SemaphoreType.DMA,
    ],
)


def _mlp_body(x_ref, agg_ref, wa_ref, ba_ref, wb_ref, bb_ref, o_ref,
              *, final_relu):
    z = x_ref[...] + agg_ref[0] + agg_ref[1]
    h = jnp.dot(z, wa_ref[...], preferred_element_type=jnp.float32)
    h = jnp.maximum(h + ba_ref[...], 0.0)
    h = jnp.dot(h, wb_ref[...], preferred_element_type=jnp.float32)
    h = h + bb_ref[...]
    if final_relu:
        h = jnp.maximum(h, 0.0)
    o_ref[...] = h


def _mlp(x, agg, wa, ba, wb, bb, final_relu):
    rows = 1000
    grid = (N // rows,)
    k = wb.shape[1]
    return pl.pallas_call(
        functools.partial(_mlp_body, final_relu=final_relu),
        grid=grid,
        in_specs=[
            pl.BlockSpec((rows, D), lambda i: (i, 0)),
            pl.BlockSpec((NC, rows, D), lambda i: (0, i, 0)),
            pl.BlockSpec((D, D), lambda i: (0, 0)),
            pl.BlockSpec((1, D), lambda i: (0, 0)),
            pl.BlockSpec((D, k), lambda i: (0, 0)),
            pl.BlockSpec((1, k), lambda i: (0, 0)),
        ],
        out_specs=pl.BlockSpec((rows, k), lambda i: (i, 0)),
        out_shape=jax.ShapeDtypeStruct((N, k), jnp.float32),
    )(x, agg, wa, ba.reshape(1, -1), wb, bb.reshape(1, -1))


def kernel(x, edge_index, W1a, b1a, W1b, b1b, W2a, b2a, W2b, b2b):
    ei = edge_index.astype(jnp.int32)
    npad = E_PAD - E
    src = jnp.concatenate([ei[0], jnp.zeros((npad,), jnp.int32)])
    pad_dst = N + (jnp.arange(npad, dtype=jnp.int32) % (AGG_ROWS - N))
    dst = jnp.concatenate([ei[1], pad_dst])
    eidx = jnp.stack([src.reshape(NW, 2, HSTEPS, CH),
                      dst.reshape(NW, 2, HSTEPS, CH)], axis=2)
    zeros_blk = jnp.zeros((ZROWS, D), jnp.float32)

    agg1 = _sc_agg(x, eidx, zeros_blk)
    h = _mlp(x, agg1, W1a, b1a, W1b, b1b, final_relu=True)
    agg2 = _sc_agg(h, eidx, zeros_blk)
    return _mlp(h, agg2, W2a, b2a, W2b, b2b, final_relu=False)


# speedup vs baseline: 7.0479x; 2.1522x over previous
import functools

import jax
import jax.numpy as jnp
from jax import lax
from jax.experimental import pallas as pl
from jax.experimental.pallas import tpu as pltpu
from jax.experimental.pallas import tpu_sc as plsc

N = 10000
D = 128
HD = D // 2
E = 320000
NC = 2
NS = 16
NW = NC * NS
CH = 128
EPW = 10240
E_PAD = NW * EPW
STEPS = EPW // CH
HSTEPS = STEPS // 2
NBUF = 2
AGG_ROWS = 10240
ZROWS = AGG_ROWS // NS


def _sc_agg_body(x0_hbm, x1_hbm, eidx_hbm, zeros_hbm, out_hbm,
                 idx_v, rows_v, xsp, aggh, sem_g, sem_s):
    cid = lax.axis_index("c")
    sid = lax.axis_index("s")
    wid = cid * NS + sid

    def gather(i, b):
        return pltpu.async_copy(xsp.at[idx_v.at[0, i]], rows_v.at[b],
                                sem_g)

    def gather_wait(i, b):
        pltpu.make_async_copy(xsp.at[idx_v.at[0, i]], rows_v.at[b],
                              sem_g).wait()

    def scatter(i, b):
        return pltpu.async_copy(rows_v.at[b], aggh.at[idx_v.at[1, i]],
                                sem_s, add=True)

    def scatter_wait(i, b):
        pltpu.make_async_copy(rows_v.at[b], aggh.at[idx_v.at[1, i]],
                              sem_s).wait()

    for h in range(2):
        xh = x0_hbm if h == 0 else x1_hbm
        pltpu.sync_copy(xh.at[pl.ds(sid * ZROWS, ZROWS)],
                        xsp.at[pl.ds(sid * ZROWS, ZROWS)])
        pltpu.sync_copy(zeros_hbm, aggh.at[pl.ds(sid * ZROWS, ZROWS)])
        plsc.subcore_barrier()

        for hh in range(2):
            pltpu.sync_copy(eidx_hbm.at[wid, hh], idx_v)
            for j in range(NBUF - 1):
                gather(j, j)
            gather_wait(0, 0)
            scatter(0, 0)
            gather(NBUF - 1, NBUF - 1)

            def step(i, carry):
                b = lax.rem(i, NBUF)
                nb = lax.rem(i + NBUF - 1, NBUF)
                gather_wait(i, b)
                scatter(i, b)
                scatter_wait(i - 1, nb)
                gather(i + NBUF - 1, nb)
                return carry

            lax.fori_loop(1, HSTEPS - NBUF + 1, step, 0)
            for e in range(HSTEPS - NBUF + 1, HSTEPS):
                gather_wait(e, e % NBUF)
                scatter(e, e % NBUF)
                scatter_wait(e - 1, (e - 1) % NBUF)
            scatter_wait(HSTEPS - 1, (HSTEPS - 1) % NBUF)

        plsc.subcore_barrier()
        pltpu.sync_copy(aggh.at[pl.ds(sid * ZROWS, ZROWS)],
                        out_hbm.at[cid, h, pl.ds(sid * ZROWS, ZROWS)])
        plsc.subcore_barrier()


_sc_agg = pl.kernel(
    _sc_agg_body,
    out_type=jax.ShapeDtypeStruct((NC, 2, AGG_ROWS, HD), jnp.float32),
    mesh=plsc.VectorSubcoreMesh(core_axis_name="c", subcore_axis_name="s"),
    compiler_params=pltpu.CompilerParams(use_tc_tiling_on_sc=False),
    scratch_types=[
        pltpu.VMEM((2, HSTEPS, CH), jnp.int32),
        pltpu.VMEM((NBUF, CH, HD), jnp.float32),
        pltpu.VMEM_SHARED((AGG_ROWS, HD), jnp.float32),
        pltpu.VMEM_SHARED((AGG_ROWS, HD), jnp.float32),
        pltpu.SemaphoreType.DMA,
        pltpu.SemaphoreType.DMA,
    ],
)


def _mlp_body(x0_ref, x1_ref, agg_ref, wa_ref, ba_ref, wb_ref, bb_ref,
              *out_refs, split_out):
    z_lo = x0_ref[...] + agg_ref[0, 0] + agg_ref[1, 0]
    z_hi = x1_ref[...] + agg_ref[0, 1] + agg_ref[1, 1]
    z = jnp.concatenate([z_lo, z_hi], axis=1)
    h = jnp.dot(z, wa_ref[...], preferred_element_type=jnp.float32)
    h = jnp.maximum(h + ba_ref[...], 0.0)
    h = jnp.dot(h, wb_ref[...], preferred_element_type=jnp.float32)
    h = h + bb_ref[...]
    if split_out:
        h = jnp.maximum(h, 0.0)
        out_refs[0][...] = h[:, :HD]
        out_refs[1][...] = h[:, HD:]
    else:
        out_refs[0][...] = h


def _mlp(x0, x1, agg, wa, ba, wb, bb, split_out):
    rows = 1000
    grid = (N // rows,)
    k = wb.shape[1]
    if split_out:
        out_shape = [jax.ShapeDtypeStruct((AGG_ROWS, HD), jnp.float32),
                     jax.ShapeDtypeStruct((AGG_ROWS, HD), jnp.float32)]
        out_specs = [pl.BlockSpec((rows, HD), lambda i: (i, 0)),
                     pl.BlockSpec((rows, HD), lambda i: (i, 0))]
    else:
        out_shape = jax.ShapeDtypeStruct((N, k), jnp.float32)
        out_specs = pl.BlockSpec((rows, k), lambda i: (i, 0))
    return pl.pallas_call(
        functools.partial(_mlp_body, split_out=split_out),
        grid=grid,
        in_specs=[
            pl.BlockSpec((rows, HD), lambda i: (i, 0)),
            pl.BlockSpec((rows, HD), lambda i: (i, 0)),
            pl.BlockSpec((NC, 2, rows, HD), lambda i: (0, 0, i, 0)),
            pl.BlockSpec((D, D), lambda i: (0, 0)),
            pl.BlockSpec((1, D), lambda i: (0, 0)),
            pl.BlockSpec((D, k), lambda i: (0, 0)),
            pl.BlockSpec((1, k), lambda i: (0, 0)),
        ],
        out_specs=out_specs,
        out_shape=out_shape,
    )(x0, x1, agg, wa, ba.reshape(1, -1), wb, bb.reshape(1, -1))


def kernel(x, edge_index, W1a, b1a, W1b, b1b, W2a, b2a, W2b, b2b):
    ei = edge_index.astype(jnp.int32)
    npad = E_PAD - E
    src = jnp.concatenate([ei[0], jnp.zeros((npad,), jnp.int32)])
    pad_dst = N + (jnp.arange(npad, dtype=jnp.int32) % (AGG_ROWS - N))
    dst = jnp.concatenate([ei[1], pad_dst])
    eidx = jnp.stack([src.reshape(NW, 2, HSTEPS, CH),
                      dst.reshape(NW, 2, HSTEPS, CH)], axis=2)
    zeros_blk = jnp.zeros((ZROWS, HD), jnp.float32)
    rpad = jnp.zeros((AGG_ROWS - N, HD), jnp.float32)
    x0 = jnp.concatenate([x[:, :HD], rpad])
    x1 = jnp.concatenate([x[:, HD:], rpad])

    agg1 = _sc_agg(x0, x1, eidx, zeros_blk)
    h0, h1 = _mlp(x0, x1, agg1, W1a, b1a, W1b, b1b, split_out=True)
    agg2 = _sc_agg(h0, h1, eidx, zeros_blk)
    return _mlp(h0, h1, agg2, W2a, b2a, W2b, b2b, split_out=False)


# speedup vs baseline: 7.5993x; 1.0782x over previous
import functools

import jax
import jax.numpy as jnp
from jax import lax
from jax.experimental import pallas as pl
from jax.experimental.pallas import tpu as pltpu
from jax.experimental.pallas import tpu_sc as plsc

N = 10000
D = 128
HD = D // 2
E = 320000
NC = 2
NS = 16
NW = NC * NS
CH = 128
EPW = 10240
E_PAD = NW * EPW
STEPS = EPW // CH
HSTEPS = STEPS // 2
NBUF = 4
AGG_ROWS = 10240
ZROWS = AGG_ROWS // NS


def _sc_agg_body(x0_hbm, x1_hbm, eidx_hbm, zeros_hbm, out_hbm,
                 idx_v, rows_v, xsp, aggh, sem_g, sem_s):
    cid = lax.axis_index("c")
    sid = lax.axis_index("s")
    wid = cid * NS + sid

    def gather(i, b):
        return pltpu.async_copy(xsp.at[idx_v.at[0, i]], rows_v.at[b],
                                sem_g)

    def gather_wait(i, b):
        pltpu.make_async_copy(xsp.at[idx_v.at[0, i]], rows_v.at[b],
                              sem_g).wait()

    def scatter(i, b):
        return pltpu.async_copy(rows_v.at[b], aggh.at[idx_v.at[1, i]],
                                sem_s, add=True)

    def scatter_wait(i, b):
        pltpu.make_async_copy(rows_v.at[b], aggh.at[idx_v.at[1, i]],
                              sem_s).wait()

    for h in range(2):
        xh = x0_hbm if h == 0 else x1_hbm
        pltpu.sync_copy(xh.at[pl.ds(sid * ZROWS, ZROWS)],
                        xsp.at[pl.ds(sid * ZROWS, ZROWS)])
        pltpu.sync_copy(zeros_hbm, aggh.at[pl.ds(sid * ZROWS, ZROWS)])
        plsc.subcore_barrier()

        for hh in range(2):
            pltpu.sync_copy(eidx_hbm.at[wid, hh], idx_v)
            for j in range(NBUF - 1):
                gather(j, j)
            gather_wait(0, 0)
            scatter(0, 0)
            gather(NBUF - 1, NBUF - 1)

            def step(i, carry):
                b = lax.rem(i, NBUF)
                nb = lax.rem(i + NBUF - 1, NBUF)
                gather_wait(i, b)
                scatter(i, b)
                scatter_wait(i - 1, nb)
                gather(i + NBUF - 1, nb)
                return carry

            lax.fori_loop(1, HSTEPS - NBUF + 1, step, 0)
            for e in range(HSTEPS - NBUF + 1, HSTEPS):
                gather_wait(e, e % NBUF)
                scatter(e, e % NBUF)
                scatter_wait(e - 1, (e - 1) % NBUF)
            scatter_wait(HSTEPS - 1, (HSTEPS - 1) % NBUF)

        plsc.subcore_barrier()
        pltpu.sync_copy(aggh.at[pl.ds(sid * ZROWS, ZROWS)],
                        out_hbm.at[cid, h, pl.ds(sid * ZROWS, ZROWS)])
        plsc.subcore_barrier()


_sc_agg = pl.kernel(
    _sc_agg_body,
    out_type=jax.ShapeDtypeStruct((NC, 2, AGG_ROWS, HD), jnp.float32),
    mesh=plsc.VectorSubcoreMesh(core_axis_name="c", subcore_axis_name="s"),
    compiler_params=pltpu.CompilerParams(use_tc_tiling_on_sc=False),
    scratch_types=[
        pltpu.VMEM((2, HSTEPS, CH), jnp.int32),
        pltpu.VMEM((NBUF, CH, HD), jnp.float32),
        pltpu.VMEM_SHARED((AGG_ROWS, HD), jnp.float32),
        pltpu.VMEM_SHARED((AGG_ROWS, HD), jnp.float32),
        pltpu.SemaphoreType.DMA,
        pltpu.SemaphoreType.DMA,
    ],
)


def _mlp_body(x0_ref, x1_ref, agg_ref, wa_ref, ba_ref, wb_ref, bb_ref,
              *out_refs, split_out):
    z_lo = x0_ref[...] + agg_ref[0, 0] + agg_ref[1, 0]
    z_hi = x1_ref[...] + agg_ref[0, 1] + agg_ref[1, 1]
    z = jnp.concatenate([z_lo, z_hi], axis=1)
    h = jnp.dot(z, wa_ref[...], preferred_element_type=jnp.float32)
    h = jnp.maximum(h + ba_ref[...], 0.0)
    h = jnp.dot(h, wb_ref[...], preferred_element_type=jnp.float32)
    h = h + bb_ref[...]
    if split_out:
        h = jnp.maximum(h, 0.0)
        out_refs[0][...] = h[:, :HD]
        out_refs[1][...] = h[:, HD:]
    else:
        out_refs[0][...] = h


def _mlp(x0, x1, agg, wa, ba, wb, bb, split_out):
    rows = 1000
    grid = (N // rows,)
    k = wb.shape[1]
    if split_out:
        out_shape = [jax.ShapeDtypeStruct((AGG_ROWS, HD), jnp.float32),
                     jax.ShapeDtypeStruct((AGG_ROWS, HD), jnp.float32)]
        out_specs = [pl.BlockSpec((rows, HD), lambda i: (i, 0)),
                     pl.BlockSpec((rows, HD), lambda i: (i, 0))]
    else:
        out_shape = jax.ShapeDtypeStruct((N, k), jnp.float32)
        out_specs = pl.BlockSpec((rows, k), lambda i: (i, 0))
    return pl.pallas_call(
        functools.partial(_mlp_body, split_out=split_out),
        grid=grid,
        in_specs=[
            pl.BlockSpec((rows, HD), lambda i: (i, 0)),
            pl.BlockSpec((rows, HD), lambda i: (i, 0)),
            pl.BlockSpec((NC, 2, rows, HD), lambda i: (0, 0, i, 0)),
            pl.BlockSpec((D, D), lambda i: (0, 0)),
            pl.BlockSpec((1, D), lambda i: (0, 0)),
            pl.BlockSpec((D, k), lambda i: (0, 0)),
            pl.BlockSpec((1, k), lambda i: (0, 0)),
        ],
        out_specs=out_specs,
        out_shape=out_shape,
    )(x0, x1, agg, wa, ba.reshape(1, -1), wb, bb.reshape(1, -1))


def kernel(x, edge_index, W1a, b1a, W1b, b1b, W2a, b2a, W2b, b2b):
    ei = edge_index.astype(jnp.int32)
    npad = E_PAD - E
    src = jnp.concatenate([ei[0], jnp.zeros((npad,), jnp.int32)])
    pad_dst = N + (jnp.arange(npad, dtype=jnp.int32) % (AGG_ROWS - N))
    dst = jnp.concatenate([ei[1], pad_dst])
    eidx = jnp.stack([src.reshape(NW, 2, HSTEPS, CH),
                      dst.reshape(NW, 2, HSTEPS, CH)], axis=2)
    zeros_blk = jnp.zeros((ZROWS, HD), jnp.float32)
    rpad = jnp.zeros((AGG_ROWS - N, HD), jnp.float32)
    x0 = jnp.concatenate([x[:, :HD], rpad])
    x1 = jnp.concatenate([x[:, HD:], rpad])

    agg1 = _sc_agg(x0, x1, eidx, zeros_blk)
    h0, h1 = _mlp(x0, x1, agg1, W1a, b1a, W1b, b1b, split_out=True)
    agg2 = _sc_agg(h0, h1, eidx, zeros_blk)
    return _mlp(h0, h1, agg2, W2a, b2a, W2b, b2b, split_out=False)


# speedup vs baseline: 7.6460x; 1.0061x over previous
import functools

import jax
import jax.numpy as jnp
from jax import lax
from jax.experimental import pallas as pl
from jax.experimental.pallas import tpu as pltpu
from jax.experimental.pallas import tpu_sc as plsc

N = 10000
D = 128
HD = D // 2
E = 320000
NC = 2
NS = 16
NW = NC * NS
CH = 128
EPW = 10240
E_PAD = NW * EPW
STEPS = EPW // CH
NBUF = 3
AGG_ROWS = 10240
ZROWS = AGG_ROWS // NS


def _sc_agg_body(x0_hbm, x1_hbm, eidx_hbm, zeros_hbm, out_hbm,
                 idx_v, rows_v, xsp, aggh, sem_g, sem_s):
    cid = lax.axis_index("c")
    sid = lax.axis_index("s")
    wid = cid * NS + sid

    def gather(i, b):
        return pltpu.async_copy(xsp.at[idx_v.at[0, i]], rows_v.at[b],
                                sem_g)

    def gather_wait(i, b):
        pltpu.make_async_copy(xsp.at[idx_v.at[0, i]], rows_v.at[b],
                              sem_g).wait()

    def scatter(i, b):
        return pltpu.async_copy(rows_v.at[b], aggh.at[idx_v.at[1, i]],
                                sem_s, add=True)

    def scatter_wait(i, b):
        pltpu.make_async_copy(rows_v.at[b], aggh.at[idx_v.at[1, i]],
                              sem_s).wait()

    for h in range(2):
        xh = x0_hbm if h == 0 else x1_hbm
        pltpu.sync_copy(xh.at[pl.ds(sid * ZROWS, ZROWS)],
                        xsp.at[pl.ds(sid * ZROWS, ZROWS)])
        pltpu.sync_copy(zeros_hbm, aggh.at[pl.ds(sid * ZROWS, ZROWS)])
        plsc.subcore_barrier()

        pltpu.sync_copy(eidx_hbm.at[wid], idx_v)
        for j in range(NBUF - 1):
            gather(j, j)
        gather_wait(0, 0)
        scatter(0, 0)
        gather(NBUF - 1, NBUF - 1)

        def step(i, carry):
            b = lax.rem(i, NBUF)
            nb = lax.rem(i + NBUF - 1, NBUF)
            gather_wait(i, b)
            scatter(i, b)
            scatter_wait(i - 1, nb)
            gather(i + NBUF - 1, nb)
            return carry

        lax.fori_loop(1, STEPS - NBUF + 1, step, 0)
        for e in range(STEPS - NBUF + 1, STEPS):
            gather_wait(e, e % NBUF)
            scatter(e, e % NBUF)
            scatter_wait(e - 1, (e - 1) % NBUF)
        scatter_wait(STEPS - 1, (STEPS - 1) % NBUF)

        plsc.subcore_barrier()
        pltpu.sync_copy(aggh.at[pl.ds(sid * ZROWS, ZROWS)],
                        out_hbm.at[cid, h, pl.ds(sid * ZROWS, ZROWS)])
        plsc.subcore_barrier()


_sc_agg = pl.kernel(
    _sc_agg_body,
    out_type=jax.ShapeDtypeStruct((NC, 2, AGG_ROWS, HD), jnp.float32),
    mesh=plsc.VectorSubcoreMesh(core_axis_name="c", subcore_axis_name="s"),
    compiler_params=pltpu.CompilerParams(use_tc_tiling_on_sc=False),
    scratch_types=[
        pltpu.VMEM((2, STEPS, CH), jnp.int32),
        pltpu.VMEM((NBUF, CH, HD), jnp.float32),
        pltpu.VMEM_SHARED((AGG_ROWS, HD), jnp.float32),
        pltpu.VMEM_SHARED((AGG_ROWS, HD), jnp.float32),
        pltpu.SemaphoreType.DMA,
        pltpu.SemaphoreType.DMA,
    ],
)


def _mlp_body(x0_ref, x1_ref, agg_ref, wa_ref, ba_ref, wb_ref, bb_ref,
              *out_refs, split_out):
    z_lo = x0_ref[...] + agg_ref[0, 0] + agg_ref[1, 0]
    z_hi = x1_ref[...] + agg_ref[0, 1] + agg_ref[1, 1]
    z = jnp.concatenate([z_lo, z_hi], axis=1)
    h = jnp.dot(z, wa_ref[...], preferred_element_type=jnp.float32)
    h = jnp.maximum(h + ba_ref[...], 0.0)
    h = jnp.dot(h, wb_ref[...], preferred_element_type=jnp.float32)
    h = h + bb_ref[...]
    if split_out:
        h = jnp.maximum(h, 0.0)
        out_refs[0][...] = h[:, :HD]
        out_refs[1][...] = h[:, HD:]
    else:
        out_refs[0][...] = h


def _mlp(x0, x1, agg, wa, ba, wb, bb, split_out):
    rows = 1000
    grid = (N // rows,)
    k = wb.shape[1]
    if split_out:
        out_shape = [jax.ShapeDtypeStruct((AGG_ROWS, HD), jnp.float32),
                     jax.ShapeDtypeStruct((AGG_ROWS, HD), jnp.float32)]
        out_specs = [pl.BlockSpec((rows, HD), lambda i: (i, 0)),
                     pl.BlockSpec((rows, HD), lambda i: (i, 0))]
    else:
        out_shape = jax.ShapeDtypeStruct((N, k), jnp.float32)
        out_specs = pl.BlockSpec((rows, k), lambda i: (i, 0))
    return pl.pallas_call(
        functools.partial(_mlp_body, split_out=split_out),
        grid=grid,
        in_specs=[
            pl.BlockSpec((rows, HD), lambda i: (i, 0)),
            pl.BlockSpec((rows, HD), lambda i: (i, 0)),
            pl.BlockSpec((NC, 2, rows, HD), lambda i: (0, 0, i, 0)),
            pl.BlockSpec((D, D), lambda i: (0, 0)),
            pl.BlockSpec((1, D), lambda i: (0, 0)),
            pl.BlockSpec((D, k), lambda i: (0, 0)),
            pl.BlockSpec((1, k), lambda i: (0, 0)),
        ],
        out_specs=out_specs,
        out_shape=out_shape,
    )(x0, x1, agg, wa, ba.reshape(1, -1), wb, bb.reshape(1, -1))


def kernel(x, edge_index, W1a, b1a, W1b, b1b, W2a, b2a, W2b, b2b):
    ei = edge_index.astype(jnp.int32)
    npad = E_PAD - E
    src = jnp.concatenate([ei[0], jnp.zeros((npad,), jnp.int32)])
    pad_dst = N + (jnp.arange(npad, dtype=jnp.int32) % (AGG_ROWS - N))
    dst = jnp.concatenate([ei[1], pad_dst])
    eidx = jnp.stack([src.reshape(NW, STEPS, CH),
                      dst.reshape(NW, STEPS, CH)], axis=1)
    zeros_blk = jnp.zeros((ZROWS, HD), jnp.float32)
    rpad = jnp.zeros((AGG_ROWS - N, HD), jnp.float32)
    x0 = jnp.concatenate([x[:, :HD], rpad])
    x1 = jnp.concatenate([x[:, HD:], rpad])

    agg1 = _sc_agg(x0, x1, eidx, zeros_blk)
    h0, h1 = _mlp(x0, x1, agg1, W1a, b1a, W1b, b1b, split_out=True)
    agg2 = _sc_agg(h0, h1, eidx, zeros_blk)
    return _mlp(h0, h1, agg2, W2a, b2a, W2b, b2b, split_out=False)


# speedup vs baseline: 7.7014x; 1.0072x over previous
import functools

import jax
import jax.numpy as jnp
from jax import lax
from jax.experimental import pallas as pl
from jax.experimental.pallas import tpu as pltpu
from jax.experimental.pallas import tpu_sc as plsc

N = 10000
D = 128
HD = D // 2
E = 320000
NC = 2
NS = 16
NW = NC * NS
CH = 128
EPW = 10240
E_PAD = NW * EPW
STEPS = EPW // CH
NBUF = 3
KH = 14
AGG_ROWS = 10240
ZROWS = AGG_ROWS // NS


def _sc_agg_body(x0_hbm, x1_hbm, eidx_hbm, zeros_hbm, out_hbm,
                 idx_v, rows_v, xsp, aggh, sem_g, sem_s):
    cid = lax.axis_index("c")
    sid = lax.axis_index("s")
    wid = cid * NS + sid

    def gather(i, b, src):
        return pltpu.async_copy(src.at[idx_v.at[0, i]], rows_v.at[b],
                                sem_g)

    def gather_wait(i, b):
        pltpu.make_async_copy(xsp.at[idx_v.at[0, i]], rows_v.at[b],
                              sem_g).wait()

    def scatter(i, b):
        return pltpu.async_copy(rows_v.at[b], aggh.at[idx_v.at[1, i]],
                                sem_s, add=True)

    def scatter_wait(i, b):
        pltpu.make_async_copy(rows_v.at[b], aggh.at[idx_v.at[1, i]],
                              sem_s).wait()

    for h in range(2):
        xh = x0_hbm if h == 0 else x1_hbm
        pltpu.sync_copy(xh.at[pl.ds(sid * ZROWS, ZROWS)],
                        xsp.at[pl.ds(sid * ZROWS, ZROWS)])
        pltpu.sync_copy(zeros_hbm, aggh.at[pl.ds(sid * ZROWS, ZROWS)])
        plsc.subcore_barrier()

        pltpu.sync_copy(eidx_hbm.at[wid], idx_v)
        for j in range(NBUF - 1):
            gather(j, j, xh)
        gather_wait(0, 0)
        scatter(0, 0)
        gather(NBUF - 1, NBUF - 1, xh)

        def make_step(src):
            def step(i, carry):
                b = lax.rem(i, NBUF)
                nb = lax.rem(i + NBUF - 1, NBUF)
                gather_wait(i, b)
                scatter(i, b)
                scatter_wait(i - 1, nb)
                gather(i + NBUF - 1, nb, src)
                return carry
            return step

        lax.fori_loop(1, KH - NBUF + 1, make_step(xh), 0)
        lax.fori_loop(KH - NBUF + 1, STEPS - NBUF + 1, make_step(xsp), 0)
        for e in range(STEPS - NBUF + 1, STEPS):
            gather_wait(e, e % NBUF)
            scatter(e, e % NBUF)
            scatter_wait(e - 1, (e - 1) % NBUF)
        scatter_wait(STEPS - 1, (STEPS - 1) % NBUF)

        plsc.subcore_barrier()
        pltpu.sync_copy(aggh.at[pl.ds(sid * ZROWS, ZROWS)],
                        out_hbm.at[cid, h, pl.ds(sid * ZROWS, ZROWS)])
        plsc.subcore_barrier()


_sc_agg = pl.kernel(
    _sc_agg_body,
    out_type=jax.ShapeDtypeStruct((NC, 2, AGG_ROWS, HD), jnp.float32),
    mesh=plsc.VectorSubcoreMesh(core_axis_name="c", subcore_axis_name="s"),
    compiler_params=pltpu.CompilerParams(use_tc_tiling_on_sc=False),
    scratch_types=[
        pltpu.VMEM((2, STEPS, CH), jnp.int32),
        pltpu.VMEM((NBUF, CH, HD), jnp.float32),
        pltpu.VMEM_SHARED((AGG_ROWS, HD), jnp.float32),
        pltpu.VMEM_SHARED((AGG_ROWS, HD), jnp.float32),
        pltpu.SemaphoreType.DMA,
        pltpu.SemaphoreType.DMA,
    ],
)


def _mlp_body(x0_ref, x1_ref, agg_ref, wa_ref, ba_ref, wb_ref, bb_ref,
              *out_refs, split_out):
    z_lo = x0_ref[...] + agg_ref[0, 0] + agg_ref[1, 0]
    z_hi = x1_ref[...] + agg_ref[0, 1] + agg_ref[1, 1]
    z = jnp.concatenate([z_lo, z_hi], axis=1)
    h = jnp.dot(z, wa_ref[...], preferred_element_type=jnp.float32)
    h = jnp.maximum(h + ba_ref[...], 0.0)
    h = jnp.dot(h, wb_ref[...], preferred_element_type=jnp.float32)
    h = h + bb_ref[...]
    if split_out:
        h = jnp.maximum(h, 0.0)
        out_refs[0][...] = h[:, :HD]
        out_refs[1][...] = h[:, HD:]
    else:
        out_refs[0][...] = h


def _mlp(x0, x1, agg, wa, ba, wb, bb, split_out):
    rows = 1000
    grid = (N // rows,)
    k = wb.shape[1]
    if split_out:
        out_shape = [jax.ShapeDtypeStruct((AGG_ROWS, HD), jnp.float32),
                     jax.ShapeDtypeStruct((AGG_ROWS, HD), jnp.float32)]
        out_specs = [pl.BlockSpec((rows, HD), lambda i: (i, 0)),
                     pl.BlockSpec((rows, HD), lambda i: (i, 0))]
    else:
        out_shape = jax.ShapeDtypeStruct((N, k), jnp.float32)
        out_specs = pl.BlockSpec((rows, k), lambda i: (i, 0))
    return pl.pallas_call(
        functools.partial(_mlp_body, split_out=split_out),
        grid=grid,
        in_specs=[
            pl.BlockSpec((rows, HD), lambda i: (i, 0)),
            pl.BlockSpec((rows, HD), lambda i: (i, 0)),
            pl.BlockSpec((NC, 2, rows, HD), lambda i: (0, 0, i, 0)),
            pl.BlockSpec((D, D), lambda i: (0, 0)),
            pl.BlockSpec((1, D), lambda i: (0, 0)),
            pl.BlockSpec((D, k), lambda i: (0, 0)),
            pl.BlockSpec((1, k), lambda i: (0, 0)),
        ],
        out_specs=out_specs,
        out_shape=out_shape,
    )(x0, x1, agg, wa, ba.reshape(1, -1), wb, bb.reshape(1, -1))


def kernel(x, edge_index, W1a, b1a, W1b, b1b, W2a, b2a, W2b, b2b):
    ei = edge_index.astype(jnp.int32)
    npad = E_PAD - E
    src = jnp.concatenate([ei[0], jnp.zeros((npad,), jnp.int32)])
    pad_dst = N + (jnp.arange(npad, dtype=jnp.int32) % (AGG_ROWS - N))
    dst = jnp.concatenate([ei[1], pad_dst])
    eidx = jnp.stack([src.reshape(NW, STEPS, CH),
                      dst.reshape(NW, STEPS, CH)], axis=1)
    zeros_blk = jnp.zeros((ZROWS, HD), jnp.float32)
    rpad = jnp.zeros((AGG_ROWS - N, HD), jnp.float32)
    x0 = jnp.concatenate([x[:, :HD], rpad])
    x1 = jnp.concatenate([x[:, HD:], rpad])

    agg1 = _sc_agg(x0, x1, eidx, zeros_blk)
    h0, h1 = _mlp(x0, x1, agg1, W1a, b1a, W1b, b1b, split_out=True)
    agg2 = _sc_agg(h0, h1, eidx, zeros_blk)
    return _mlp(h0, h1, agg2, W2a, b2a, W2b, b2b, split_out=False)
